# baseline (speedup 1.0000x reference)
import jax
import jax.numpy as jnp
from jax.experimental import pallas as pl
from jax.experimental.pallas import tpu as pltpu

DIM = 768
NUM_EXPERTS = 8
TOP_K = 2
NUM_TOKENS = 2048


def _moe_kernel(x_ref, wg_ref, we_ref, o_ref, i1_ref, i2_ref, w1_ref, w2_ref):
    e = pl.program_id(0)
    x = x_ref[...]

    @pl.when(e == 0)
    def _gate():
        logits = jnp.dot(x, wg_ref[...], preferred_element_type=jnp.float32)
        iota = jax.lax.broadcasted_iota(jnp.int32, logits.shape, 1)
        v1 = jnp.max(logits, axis=1, keepdims=True)
        i1 = jnp.min(jnp.where(logits == v1, iota, NUM_EXPERTS), axis=1, keepdims=True)
        l2 = jnp.where(iota == i1, -jnp.inf, logits)
        v2 = jnp.max(l2, axis=1, keepdims=True)
        i2 = jnp.min(jnp.where(l2 == v2, iota, NUM_EXPERTS), axis=1, keepdims=True)
        w1 = 1.0 / (1.0 + jnp.exp(v2 - v1))
        i1_ref[...] = i1
        i2_ref[...] = i2
        w1_ref[...] = w1
        w2_ref[...] = 1.0 - w1

    scale = jnp.where(i1_ref[...] == e, w1_ref[...], 0.0) + jnp.where(
        i2_ref[...] == e, w2_ref[...], 0.0
    )
    y = jnp.dot(x, we_ref[0], preferred_element_type=jnp.float32)

    @pl.when(e == 0)
    def _first():
        o_ref[...] = scale * y

    @pl.when(e != 0)
    def _rest():
        o_ref[...] += scale * y


def kernel(inputs, Wg, We):
    return pl.pallas_call(
        _moe_kernel,
        grid=(NUM_EXPERTS,),
        in_specs=[
            pl.BlockSpec((NUM_TOKENS, DIM), lambda e: (0, 0)),
            pl.BlockSpec((DIM, NUM_EXPERTS), lambda e: (0, 0)),
            pl.BlockSpec((1, DIM, DIM), lambda e: (e, 0, 0)),
        ],
        out_specs=pl.BlockSpec((NUM_TOKENS, DIM), lambda e: (0, 0)),
        out_shape=jax.ShapeDtypeStruct((NUM_TOKENS, DIM), jnp.float32),
        scratch_shapes=[
            pltpu.VMEM((NUM_TOKENS, 1), jnp.int32),
            pltpu.VMEM((NUM_TOKENS, 1), jnp.int32),
            pltpu.VMEM((NUM_TOKENS, 1), jnp.float32),
            pltpu.VMEM((NUM_TOKENS, 1), jnp.float32),
        ],
    )(inputs, Wg, We)


# speedup vs baseline: 1.0026x; 1.0026x over previous
import functools
import jax
import jax.numpy as jnp
from jax import lax
from jax.experimental import pallas as pl
from jax.experimental.pallas import tpu as pltpu

USE_SC = True

DIM = 768
E = 8
K = 2
T = 2048
BLK = 128
NB = 40
CAP = NB * BLK
NW = 32
TPW = T // NW


def _gate_kernel(x_ref, wg_ref, pos0_ref, pos1_ref, w0_ref, w1_ref, gid_ref):
    x = x_ref[...]
    logits = jnp.dot(x, wg_ref[...], preferred_element_type=jnp.float32)
    iota_e = jax.lax.broadcasted_iota(jnp.int32, logits.shape, 1)
    v1 = jnp.max(logits, axis=1, keepdims=True)
    i1 = jnp.min(jnp.where(logits == v1, iota_e, E), axis=1, keepdims=True)
    l2 = jnp.where(iota_e == i1, -jnp.inf, logits)
    v2 = jnp.max(l2, axis=1, keepdims=True)
    i2 = jnp.min(jnp.where(l2 == v2, iota_e, E), axis=1, keepdims=True)
    wa = 1.0 / (1.0 + jnp.exp(v2 - v1))
    w0_ref[...] = wa
    w1_ref[...] = 1.0 - wa

    one1 = (i1 == iota_e).astype(jnp.float32)
    one2 = (i2 == iota_e).astype(jnp.float32)
    cnt = one1 + one2
    r = jax.lax.broadcasted_iota(jnp.int32, (T, T), 0)
    c = jax.lax.broadcasted_iota(jnp.int32, (T, T), 1)
    tri = (c < r).astype(jnp.float32)
    prev = jnp.dot(tri, cnt, preferred_element_type=jnp.float32)
    totals = prev[T - 1 : T, :] + cnt[T - 1 : T, :]
    ti = totals.astype(jnp.int32)
    padded = (((ti + (BLK - 1)) // BLK) * BLK).astype(jnp.float32)

    bstart = (
        jax.lax.broadcasted_iota(jnp.int32, (8, NB), 1).astype(jnp.float32)
        * float(BLK)
    )
    gid_acc = jnp.zeros((8, NB), jnp.float32)
    base_cols = []
    run = jnp.zeros((1, 1), jnp.float32)
    for ee in range(E):
        base_cols.append(run)
        run = run + padded[:, ee : ee + 1]
        if ee < E - 1:
            gid_acc = gid_acc + (bstart >= run).astype(jnp.float32)
    base_col = jnp.concatenate(base_cols, axis=0)
    gid_ref[...] = gid_acc.astype(jnp.int32)

    rank1 = jnp.sum(one1 * prev, axis=1, keepdims=True)
    rank2 = jnp.sum(one2 * prev, axis=1, keepdims=True) + jnp.sum(
        one2 * one1, axis=1, keepdims=True
    )
    b1 = jnp.dot(one1, base_col, preferred_element_type=jnp.float32)
    b2 = jnp.dot(one2, base_col, preferred_element_type=jnp.float32)
    pos0_ref[...] = (b1 + rank1).astype(jnp.int32)
    pos1_ref[...] = (b2 + rank2).astype(jnp.int32)


def _gate(x, Wg):
    return pl.pallas_call(
        _gate_kernel,
        out_shape=[
            jax.ShapeDtypeStruct((T, 1), jnp.int32),
            jax.ShapeDtypeStruct((T, 1), jnp.int32),
            jax.ShapeDtypeStruct((T, 1), jnp.float32),
            jax.ShapeDtypeStruct((T, 1), jnp.float32),
            jax.ShapeDtypeStruct((8, NB), jnp.int32),
        ],
    )(x, Wg)


def _gmm_kernel(gid_ref, xd_ref, wrow_ref, we_ref, ys_ref):
    ys_ref[...] = wrow_ref[...] * jnp.dot(
        xd_ref[...], we_ref[0], preferred_element_type=jnp.float32
    )


def _gmm(xd, wrow, We, gid):
    grid_spec = pltpu.PrefetchScalarGridSpec(
        num_scalar_prefetch=1,
        grid=(NB,),
        in_specs=[
            pl.BlockSpec((BLK, DIM), lambda b, gid: (b, 0)),
            pl.BlockSpec((BLK, 1), lambda b, gid: (b, 0)),
            pl.BlockSpec((1, DIM, DIM), lambda b, gid: (gid[b], 0, 0)),
        ],
        out_specs=pl.BlockSpec((BLK, DIM), lambda b, gid: (b, 0)),
    )
    return pl.pallas_call(
        _gmm_kernel,
        grid_spec=grid_spec,
        out_shape=jax.ShapeDtypeStruct((CAP, DIM), jnp.float32),
    )(gid, xd, wrow.reshape(CAP, 1), We)


try:
    from jax.experimental.pallas import tpu_sc as plsc

    _mesh = plsc.VectorSubcoreMesh(core_axis_name="c", subcore_axis_name="s")

    @functools.partial(
        pl.kernel,
        mesh=_mesh,
        out_type=[
            jax.ShapeDtypeStruct((CAP, DIM), jnp.float32),
            jax.ShapeDtypeStruct((CAP,), jnp.float32),
        ],
        scratch_types=[
            pltpu.VMEM((TPW, DIM), jnp.float32),
            pltpu.VMEM((TPW,), jnp.int32),
            pltpu.VMEM((TPW,), jnp.int32),
            pltpu.VMEM((TPW,), jnp.float32),
            pltpu.VMEM((TPW,), jnp.float32),
            pltpu.SemaphoreType.DMA,
        ],
    )
    def _dispatch(
        x_hbm, pos0_hbm, pos1_hbm, w0_hbm, w1_hbm,
        xd_hbm, wrow_hbm,
        rows_v, i0_v, i1_v, w0_v, w1_v, sem,
    ):
        w = lax.axis_index("s") * 2 + lax.axis_index("c")
        base = w * TPW
        pltpu.sync_copy(x_hbm.at[pl.ds(base, TPW)], rows_v)
        pltpu.sync_copy(pos0_hbm.at[pl.ds(base, TPW)], i0_v)
        pltpu.sync_copy(pos1_hbm.at[pl.ds(base, TPW)], i1_v)
        pltpu.sync_copy(w0_hbm.at[pl.ds(base, TPW)], w0_v)
        pltpu.sync_copy(w1_hbm.at[pl.ds(base, TPW)], w1_v)
        pltpu.async_copy(rows_v, xd_hbm.at[i0_v], sem).wait()
        pltpu.async_copy(rows_v, xd_hbm.at[i1_v], sem).wait()
        pltpu.async_copy(w0_v, wrow_hbm.at[i0_v], sem).wait()
        pltpu.async_copy(w1_v, wrow_hbm.at[i1_v], sem).wait()

    @functools.partial(
        pl.kernel,
        mesh=_mesh,
        out_type=jax.ShapeDtypeStruct((T, DIM), jnp.float32),
        scratch_types=[
            pltpu.VMEM((TPW, DIM), jnp.float32),
            pltpu.VMEM((TPW, DIM), jnp.float32),
            pltpu.VMEM((TPW,), jnp.int32),
            pltpu.VMEM((TPW,), jnp.int32),
            pltpu.SemaphoreType.DMA,
            pltpu.SemaphoreType.DMA,
        ],
    )
    def _combine(ys_hbm, pos0_hbm, pos1_hbm, out_hbm, r0_v, r1_v, i0_v, i1_v, sem0, sem1):
        w = lax.axis_index("s") * 2 + lax.axis_index("c")
        base = w * TPW
        pltpu.sync_copy(pos0_hbm.at[pl.ds(base, TPW)], i0_v)
        pltpu.sync_copy(pos1_hbm.at[pl.ds(base, TPW)], i1_v)
        cp0 = pltpu.async_copy(ys_hbm.at[i0_v], r0_v, sem0)
        cp1 = pltpu.async_copy(ys_hbm.at[i1_v], r1_v, sem1)
        cp0.wait()
        cp1.wait()

        def body(i, carry):
            for j in range(DIM // 16):
                sl = pl.ds(j * 16, 16)
                r0_v[i, sl] = r0_v[i, sl] + r1_v[i, sl]
            return carry

        lax.fori_loop(0, TPW, body, 0)
        pltpu.sync_copy(r0_v, out_hbm.at[pl.ds(base, TPW)])

except Exception as _e:
    _dispatch = None
    _combine = None

if True:

    def _dispatch_jnp(x, pos0, pos1, w0, w1):
        xd = jnp.zeros((CAP, DIM), jnp.float32)
        wrow = jnp.zeros((CAP,), jnp.float32)
        p0 = pos0.reshape(-1)
        p1 = pos1.reshape(-1)
        xd = xd.at[p0].set(x).at[p1].set(x)
        wrow = wrow.at[p0].set(w0.reshape(-1)).at[p1].set(w1.reshape(-1))
        return xd, wrow

    def _combine_jnp(ys, pos0, pos1):
        return ys[pos0.reshape(-1)] + ys[pos1.reshape(-1)]


def kernel(inputs, Wg, We):
    pos0, pos1, w0, w1, gid8 = _gate(inputs, Wg)
    gid = gid8[0]
    pos0 = pos0.reshape(T)
    pos1 = pos1.reshape(T)
    w0 = w0.reshape(T)
    w1 = w1.reshape(T)
    if USE_SC:
        xd, wrow = _dispatch(inputs, pos0, pos1, w0, w1)
        ys = _gmm(xd, wrow, We, gid)
        out = _combine(ys, pos0, pos1)
    else:
        xd, wrow = _dispatch_jnp(inputs, pos0, pos1, w0, w1)
        ys = _gmm(xd, wrow, We, gid)
        out = _combine_jnp(ys, pos0, pos1)
    return out


# speedup vs baseline: 1.0047x; 1.0021x over previous
import functools
import jax
import jax.numpy as jnp
from jax import lax
from jax.experimental import pallas as pl
from jax.experimental.pallas import tpu as pltpu

USE_SC = True

DIM = 768
E = 8
K = 2
T = 2048
BLK = 128
NB = 40
CAP = NB * BLK
NW = 32
TPW = T // NW


def _gate_kernel(x_ref, wg_ref, pos0_ref, pos1_ref, w0_ref, w1_ref, gid_ref):
    x = x_ref[...]
    logits = jnp.dot(x, wg_ref[...], preferred_element_type=jnp.float32)
    iota_e = jax.lax.broadcasted_iota(jnp.int32, logits.shape, 1)
    v1 = jnp.max(logits, axis=1, keepdims=True)
    i1 = jnp.min(jnp.where(logits == v1, iota_e, E), axis=1, keepdims=True)
    l2 = jnp.where(iota_e == i1, -jnp.inf, logits)
    v2 = jnp.max(l2, axis=1, keepdims=True)
    i2 = jnp.min(jnp.where(l2 == v2, iota_e, E), axis=1, keepdims=True)
    wa = 1.0 / (1.0 + jnp.exp(v2 - v1))
    w0_ref[...] = wa
    w1_ref[...] = 1.0 - wa

    one1 = (i1 == iota_e).astype(jnp.float32)
    one2 = (i2 == iota_e).astype(jnp.float32)
    cnt = one1 + one2
    r = jax.lax.broadcasted_iota(jnp.int32, (T, T), 0)
    c = jax.lax.broadcasted_iota(jnp.int32, (T, T), 1)
    tri = (c < r).astype(jnp.float32)
    prev = jnp.dot(tri, cnt, preferred_element_type=jnp.float32)
    totals = prev[T - 1 : T, :] + cnt[T - 1 : T, :]
    ti = totals.astype(jnp.int32)
    padded = (((ti + (BLK - 1)) // BLK) * BLK).astype(jnp.float32)

    bstart = (
        jax.lax.broadcasted_iota(jnp.int32, (8, NB), 1).astype(jnp.float32)
        * float(BLK)
    )
    gid_acc = jnp.zeros((8, NB), jnp.float32)
    base_cols = []
    run = jnp.zeros((1, 1), jnp.float32)
    for ee in range(E):
        base_cols.append(run)
        run = run + padded[:, ee : ee + 1]
        if ee < E - 1:
            gid_acc = gid_acc + (bstart >= run).astype(jnp.float32)
    base_col = jnp.concatenate(base_cols, axis=0)
    gid_ref[...] = gid_acc.astype(jnp.int32)

    rank1 = jnp.sum(one1 * prev, axis=1, keepdims=True)
    rank2 = jnp.sum(one2 * prev, axis=1, keepdims=True) + jnp.sum(
        one2 * one1, axis=1, keepdims=True
    )
    b1 = jnp.dot(one1, base_col, preferred_element_type=jnp.float32)
    b2 = jnp.dot(one2, base_col, preferred_element_type=jnp.float32)
    pos0_ref[...] = (b1 + rank1).astype(jnp.int32)
    pos1_ref[...] = (b2 + rank2).astype(jnp.int32)


def _gate(x, Wg):
    return pl.pallas_call(
        _gate_kernel,
        out_shape=[
            jax.ShapeDtypeStruct((T, 1), jnp.int32),
            jax.ShapeDtypeStruct((T, 1), jnp.int32),
            jax.ShapeDtypeStruct((T, 1), jnp.float32),
            jax.ShapeDtypeStruct((T, 1), jnp.float32),
            jax.ShapeDtypeStruct((8, NB), jnp.int32),
        ],
    )(x, Wg)


def _gmm_kernel(gid_ref, xd_ref, wrow_ref, we_ref, ys_ref):
    ys_ref[...] = wrow_ref[...] * jnp.dot(
        xd_ref[...].astype(jnp.bfloat16),
        we_ref[0].astype(jnp.bfloat16),
        preferred_element_type=jnp.float32,
    )


def _gmm(xd, wrow, We, gid):
    grid_spec = pltpu.PrefetchScalarGridSpec(
        num_scalar_prefetch=1,
        grid=(NB,),
        in_specs=[
            pl.BlockSpec((BLK, DIM), lambda b, gid: (b, 0)),
            pl.BlockSpec((BLK, 1), lambda b, gid: (b, 0)),
            pl.BlockSpec((1, DIM, DIM), lambda b, gid: (gid[b], 0, 0)),
        ],
        out_specs=pl.BlockSpec((BLK, DIM), lambda b, gid: (b, 0)),
    )
    return pl.pallas_call(
        _gmm_kernel,
        grid_spec=grid_spec,
        out_shape=jax.ShapeDtypeStruct((CAP, DIM), jnp.float32),
    )(gid, xd, wrow.reshape(CAP, 1), We)


try:
    from jax.experimental.pallas import tpu_sc as plsc

    _mesh = plsc.VectorSubcoreMesh(core_axis_name="c", subcore_axis_name="s")

    @functools.partial(
        pl.kernel,
        mesh=_mesh,
        out_type=[
            jax.ShapeDtypeStruct((CAP, DIM), jnp.float32),
            jax.ShapeDtypeStruct((CAP,), jnp.float32),
        ],
        scratch_types=[
            pltpu.VMEM((TPW, DIM), jnp.float32),
            pltpu.VMEM((TPW,), jnp.int32),
            pltpu.VMEM((TPW,), jnp.int32),
            pltpu.VMEM((TPW,), jnp.float32),
            pltpu.VMEM((TPW,), jnp.float32),
            pltpu.SemaphoreType.DMA,
        ],
    )
    def _dispatch(
        x_hbm, pos0_hbm, pos1_hbm, w0_hbm, w1_hbm,
        xd_hbm, wrow_hbm,
        rows_v, i0_v, i1_v, w0_v, w1_v, sem,
    ):
        w = lax.axis_index("s") * 2 + lax.axis_index("c")
        base = w * TPW
        pltpu.sync_copy(x_hbm.at[pl.ds(base, TPW)], rows_v)
        pltpu.sync_copy(pos0_hbm.at[pl.ds(base, TPW)], i0_v)
        pltpu.sync_copy(pos1_hbm.at[pl.ds(base, TPW)], i1_v)
        pltpu.sync_copy(w0_hbm.at[pl.ds(base, TPW)], w0_v)
        pltpu.sync_copy(w1_hbm.at[pl.ds(base, TPW)], w1_v)
        pltpu.async_copy(rows_v, xd_hbm.at[i0_v], sem).wait()
        pltpu.async_copy(rows_v, xd_hbm.at[i1_v], sem).wait()
        pltpu.async_copy(w0_v, wrow_hbm.at[i0_v], sem).wait()
        pltpu.async_copy(w1_v, wrow_hbm.at[i1_v], sem).wait()

    @functools.partial(
        pl.kernel,
        mesh=_mesh,
        out_type=jax.ShapeDtypeStruct((T, DIM), jnp.float32),
        scratch_types=[
            pltpu.VMEM((TPW, DIM), jnp.float32),
            pltpu.VMEM((TPW, DIM), jnp.float32),
            pltpu.VMEM((TPW,), jnp.int32),
            pltpu.VMEM((TPW,), jnp.int32),
            pltpu.SemaphoreType.DMA,
            pltpu.SemaphoreType.DMA,
        ],
    )
    def _combine(ys_hbm, pos0_hbm, pos1_hbm, out_hbm, r0_v, r1_v, i0_v, i1_v, sem0, sem1):
        w = lax.axis_index("s") * 2 + lax.axis_index("c")
        base = w * TPW
        pltpu.sync_copy(pos0_hbm.at[pl.ds(base, TPW)], i0_v)
        pltpu.sync_copy(pos1_hbm.at[pl.ds(base, TPW)], i1_v)
        cp0 = pltpu.async_copy(ys_hbm.at[i0_v], r0_v, sem0)
        cp1 = pltpu.async_copy(ys_hbm.at[i1_v], r1_v, sem1)
        cp0.wait()
        cp1.wait()

        def body(i, carry):
            for j in range(DIM // 16):
                sl = pl.ds(j * 16, 16)
                r0_v[i, sl] = r0_v[i, sl] + r1_v[i, sl]
            return carry

        lax.fori_loop(0, TPW, body, 0)
        pltpu.sync_copy(r0_v, out_hbm.at[pl.ds(base, TPW)])

except Exception as _e:
    _dispatch = None
    _combine = None

if True:

    def _dispatch_jnp(x, pos0, pos1, w0, w1):
        xd = jnp.zeros((CAP, DIM), jnp.float32)
        wrow = jnp.zeros((CAP,), jnp.float32)
        p0 = pos0.reshape(-1)
        p1 = pos1.reshape(-1)
        xd = xd.at[p0].set(x).at[p1].set(x)
        wrow = wrow.at[p0].set(w0.reshape(-1)).at[p1].set(w1.reshape(-1))
        return xd, wrow

    def _combine_jnp(ys, pos0, pos1):
        return ys[pos0.reshape(-1)] + ys[pos1.reshape(-1)]


def kernel(inputs, Wg, We):
    pos0, pos1, w0, w1, gid8 = _gate(inputs, Wg)
    gid = gid8[0]
    pos0 = pos0.reshape(T)
    pos1 = pos1.reshape(T)
    w0 = w0.reshape(T)
    w1 = w1.reshape(T)
    if USE_SC:
        xd, wrow = _dispatch(inputs, pos0, pos1, w0, w1)
        ys = _gmm(xd, wrow, We, gid)
        out = _combine(ys, pos0, pos1)
    else:
        xd, wrow = _dispatch_jnp(inputs, pos0, pos1, w0, w1)
        ys = _gmm(xd, wrow, We, gid)
        out = _combine_jnp(ys, pos0, pos1)
    return out


# speedup vs baseline: 3.4698x; 3.4537x over previous
import jax
import jax.numpy as jnp
from jax.experimental import pallas as pl
from jax.experimental.pallas import tpu as pltpu

DIM = 768
NUM_EXPERTS = 8
TOP_K = 2
NUM_TOKENS = 2048


def _moe_kernel(x_ref, wg_ref, we_ref, o_ref, xbf_ref, i1_ref, i2_ref, w1_ref, w2_ref):
    e = pl.program_id(0)

    @pl.when(e == 0)
    def _gate():
        x = x_ref[...]
        xbf_ref[...] = x.astype(jnp.bfloat16)
        logits = jnp.dot(x, wg_ref[...], preferred_element_type=jnp.float32)
        iota = jax.lax.broadcasted_iota(jnp.int32, logits.shape, 1)
        v1 = jnp.max(logits, axis=1, keepdims=True)
        i1 = jnp.min(jnp.where(logits == v1, iota, NUM_EXPERTS), axis=1, keepdims=True)
        l2 = jnp.where(iota == i1, -jnp.inf, logits)
        v2 = jnp.max(l2, axis=1, keepdims=True)
        i2 = jnp.min(jnp.where(l2 == v2, iota, NUM_EXPERTS), axis=1, keepdims=True)
        w1 = 1.0 / (1.0 + jnp.exp(v2 - v1))
        i1_ref[...] = i1
        i2_ref[...] = i2
        w1_ref[...] = w1
        w2_ref[...] = 1.0 - w1

    scale = jnp.where(i1_ref[...] == e, w1_ref[...], 0.0) + jnp.where(
        i2_ref[...] == e, w2_ref[...], 0.0
    )
    y = jnp.dot(
        xbf_ref[...],
        we_ref[0].astype(jnp.bfloat16),
        preferred_element_type=jnp.float32,
    )

    @pl.when(e == 0)
    def _first():
        o_ref[...] = scale * y

    @pl.when(e != 0)
    def _rest():
        o_ref[...] += scale * y


def kernel(inputs, Wg, We):
    return pl.pallas_call(
        _moe_kernel,
        grid=(NUM_EXPERTS,),
        in_specs=[
            pl.BlockSpec((NUM_TOKENS, DIM), lambda e: (0, 0)),
            pl.BlockSpec((DIM, NUM_EXPERTS), lambda e: (0, 0)),
            pl.BlockSpec((1, DIM, DIM), lambda e: (e, 0, 0)),
        ],
        out_specs=pl.BlockSpec((NUM_TOKENS, DIM), lambda e: (0, 0)),
        out_shape=jax.ShapeDtypeStruct((NUM_TOKENS, DIM), jnp.float32),
        scratch_shapes=[
            pltpu.VMEM((NUM_TOKENS, DIM), jnp.bfloat16),
            pltpu.VMEM((NUM_TOKENS, 1), jnp.int32),
            pltpu.VMEM((NUM_TOKENS, 1), jnp.int32),
            pltpu.VMEM((NUM_TOKENS, 1), jnp.float32),
            pltpu.VMEM((NUM_TOKENS, 1), jnp.float32),
        ],
    )(inputs, Wg, We)


# speedup vs baseline: 3.9470x; 1.1375x over previous
import jax
import jax.numpy as jnp
from jax.experimental import pallas as pl
from jax.experimental.pallas import tpu as pltpu

DIM = 768
NUM_EXPERTS = 8
TOP_K = 2
NUM_TOKENS = 2048


def _moe_kernel(x_ref, wg_ref, we_ref, o_ref, xbf_ref, i1_ref, i2_ref, w1_ref, w2_ref):
    step = pl.program_id(0)

    @pl.when(step == 0)
    def _gate():
        x = x_ref[...]
        xbf_ref[...] = x.astype(jnp.bfloat16)
        logits = jnp.dot(x, wg_ref[...], preferred_element_type=jnp.float32)
        iota = jax.lax.broadcasted_iota(jnp.int32, logits.shape, 1)
        v1 = jnp.max(logits, axis=1, keepdims=True)
        i1 = jnp.min(jnp.where(logits == v1, iota, NUM_EXPERTS), axis=1, keepdims=True)
        l2 = jnp.where(iota == i1, -jnp.inf, logits)
        v2 = jnp.max(l2, axis=1, keepdims=True)
        i2 = jnp.min(jnp.where(l2 == v2, iota, NUM_EXPERTS), axis=1, keepdims=True)
        w1 = 1.0 / (1.0 + jnp.exp(v2 - v1))
        i1_ref[...] = i1
        i2_ref[...] = i2
        w1_ref[...] = w1
        w2_ref[...] = 1.0 - w1

    xbf = xbf_ref[...]
    i1 = i1_ref[...]
    i2 = i2_ref[...]
    w1 = w1_ref[...]
    w2 = w2_ref[...]

    def contrib(e, slot):
        scale = jnp.where(i1 == e, w1, 0.0) + jnp.where(i2 == e, w2, 0.0)
        y = jnp.dot(
            xbf, we_ref[slot].astype(jnp.bfloat16), preferred_element_type=jnp.float32
        )
        return scale * y

    ea = step * 2
    acc = contrib(ea, 0) + contrib(ea + 1, 1)

    @pl.when(step == 0)
    def _first():
        o_ref[...] = acc

    @pl.when(step != 0)
    def _rest():
        o_ref[...] += acc


def kernel(inputs, Wg, We):
    return pl.pallas_call(
        _moe_kernel,
        grid=(NUM_EXPERTS // 2,),
        in_specs=[
            pl.BlockSpec((NUM_TOKENS, DIM), lambda s: (0, 0)),
            pl.BlockSpec((DIM, NUM_EXPERTS), lambda s: (0, 0)),
            pl.BlockSpec((2, DIM, DIM), lambda s: (s, 0, 0)),
        ],
        out_specs=pl.BlockSpec((NUM_TOKENS, DIM), lambda s: (0, 0)),
        out_shape=jax.ShapeDtypeStruct((NUM_TOKENS, DIM), jnp.float32),
        scratch_shapes=[
            pltpu.VMEM((NUM_TOKENS, DIM), jnp.bfloat16),
            pltpu.VMEM((NUM_TOKENS, 1), jnp.int32),
            pltpu.VMEM((NUM_TOKENS, 1), jnp.int32),
            pltpu.VMEM((NUM_TOKENS, 1), jnp.float32),
            pltpu.VMEM((NUM_TOKENS, 1), jnp.float32),
        ],
    )(inputs, Wg, We)
